# Initial kernel scaffold; baseline (speedup 1.0000x reference)
import jax
import jax.numpy as jnp
from jax.experimental import pallas as pl


def kernel(inputs, indexes, cameras, features):
    raise NotImplementedError("write your pallas kernel here")


# speedup vs baseline: 2.9880x; 2.9880x over previous
import functools

import jax
import jax.numpy as jnp
from jax import lax
from jax.experimental import pallas as pl
from jax.experimental.pallas import tpu as pltpu, tpu_sc as plsc

_TEMP = 0.05
_B = 1024
_D = 128
_N = 100000
_TILE = 2048
_GRID = (_N + _TILE - 1) // _TILE
_CAP = 18.420680743952367


def _make_sc_gather():
    info = plsc.get_sparse_core_info()
    nc, ns = info.num_cores, info.num_subcores
    nw = nc * ns
    b_per_w = _B // nw
    mesh = plsc.VectorSubcoreMesh(core_axis_name="c", subcore_axis_name="s")

    @functools.partial(
        pl.kernel,
        mesh=mesh,
        out_type=jax.ShapeDtypeStruct((_B, _D), jnp.float32),
        scratch_types=[
            pltpu.VMEM((b_per_w,), jnp.int32),
            pltpu.VMEM((b_per_w, _D), jnp.float32),
            pltpu.SemaphoreType.DMA,
        ],
    )
    def sc_gather(table_hbm, idx_hbm, out_hbm, idx_v, rows_v, sem):
        wid = lax.axis_index("s") * nc + lax.axis_index("c")
        base = wid * b_per_w
        pltpu.sync_copy(idx_hbm.at[pl.ds(base, b_per_w)], idx_v)
        pltpu.async_copy(table_hbm.at[idx_v], rows_v, sem).wait()
        pltpu.sync_copy(rows_v, out_hbm.at[pl.ds(base, b_per_w)])

    return sc_gather


_sc_gather_cache = []


def _sc_gather(features, indexes):
    if not _sc_gather_cache:
        _sc_gather_cache.append(_make_sc_gather())
    return _sc_gather_cache[0](features, indexes)


def _tc_body(x_ref, g_ref, feat_ref, out_ref, xs_ref, m_ref, s_ref):
    i = pl.program_id(0)

    @pl.when(i == 0)
    def _init():
        x = x_ref[...]
        nrm = jnp.sqrt(jnp.sum(x * x, axis=1, keepdims=True))
        scale = 1.0 / (jnp.maximum(nrm, 1e-12) * _TEMP)
        xs_ref[...] = x * scale
        m_ref[...] = jnp.full((_B, 1), -jnp.inf, jnp.float32)
        s_ref[...] = jnp.zeros((_B, 1), jnp.float32)

    xs = xs_ref[...]
    t = lax.dot_general(
        xs, feat_ref[...], (((1,), (1,)), ((), ())),
        preferred_element_type=jnp.float32,
    )
    cols = i * _TILE + lax.broadcasted_iota(jnp.int32, (1, _TILE), 1)
    t = jnp.where(cols < _N, t, -jnp.inf)

    m_old = m_ref[...]
    m_new = jnp.maximum(m_old, jnp.max(t, axis=1, keepdims=True))
    s_new = s_ref[...] * jnp.exp(m_old - m_new) + jnp.sum(
        jnp.exp(t - m_new), axis=1, keepdims=True
    )
    m_ref[...] = m_new
    s_ref[...] = s_new

    @pl.when(i == _GRID - 1)
    def _finish():
        t_sel = jnp.sum(xs_ref[...] * g_ref[...], axis=1, keepdims=True)
        lse = m_ref[...] + jnp.log(s_ref[...])
        loss_i = jnp.minimum(lse - t_sel, _CAP)
        out_ref[...] = jnp.sum(loss_i).reshape(1, 1) * (1.0 / _B)


def _tc_loss(inputs, g, features):
    return pl.pallas_call(
        _tc_body,
        grid=(_GRID,),
        in_specs=[
            pl.BlockSpec((_B, _D), lambda i: (0, 0)),
            pl.BlockSpec((_B, _D), lambda i: (0, 0)),
            pl.BlockSpec((_TILE, _D), lambda i: (i, 0)),
        ],
        out_specs=pl.BlockSpec((1, 1), lambda i: (0, 0)),
        out_shape=jax.ShapeDtypeStruct((1, 1), jnp.float32),
        scratch_shapes=[
            pltpu.VMEM((_B, _D), jnp.float32),
            pltpu.VMEM((_B, 1), jnp.float32),
            pltpu.VMEM((_B, 1), jnp.float32),
        ],
    )(inputs, g, features)


def kernel(inputs, indexes, cameras, features):
    g = _sc_gather(features, indexes)
    out = _tc_loss(inputs, g, features)
    return out[0, 0]


# speedup vs baseline: 3.3713x; 1.1283x over previous
import functools

import jax
import jax.numpy as jnp
from jax import lax
from jax.experimental import pallas as pl
from jax.experimental.pallas import tpu as pltpu, tpu_sc as plsc

_TEMP = 0.05
_B = 1024
_D = 128
_N = 100000
_TILE = 4096
_GRID = (_N + _TILE - 1) // _TILE
_CAP = 18.420680743952367


def _make_sc_gather():
    info = plsc.get_sparse_core_info()
    nc, ns = info.num_cores, info.num_subcores
    nw = nc * ns
    b_per_w = _B // nw
    mesh = plsc.VectorSubcoreMesh(core_axis_name="c", subcore_axis_name="s")

    @functools.partial(
        pl.kernel,
        mesh=mesh,
        out_type=jax.ShapeDtypeStruct((_B, _D), jnp.float32),
        scratch_types=[
            pltpu.VMEM((b_per_w,), jnp.int32),
            pltpu.VMEM((b_per_w, _D), jnp.float32),
            pltpu.SemaphoreType.DMA,
        ],
    )
    def sc_gather(table_hbm, idx_hbm, out_hbm, idx_v, rows_v, sem):
        wid = lax.axis_index("s") * nc + lax.axis_index("c")
        base = wid * b_per_w
        pltpu.sync_copy(idx_hbm.at[pl.ds(base, b_per_w)], idx_v)
        pltpu.async_copy(table_hbm.at[idx_v], rows_v, sem).wait()
        pltpu.sync_copy(rows_v, out_hbm.at[pl.ds(base, b_per_w)])

    return sc_gather


_sc_gather_cache = []


def _sc_gather(features, indexes):
    if not _sc_gather_cache:
        _sc_gather_cache.append(_make_sc_gather())
    return _sc_gather_cache[0](features, indexes)


_LOG2E = 1.4426950408889634
_LN2 = 0.6931471805599453


def _tc_body(x_ref, g_ref, feat_ref, out_ref, xs_ref, xb_ref, s_ref):
    i = pl.program_id(0)

    @pl.when(i == 0)
    def _init():
        x = x_ref[...]
        nrm = jnp.sqrt(jnp.sum(x * x, axis=1, keepdims=True))
        scale = _LOG2E / (jnp.maximum(nrm, 1e-12) * _TEMP)
        xs = x * scale
        xs_ref[...] = xs
        xb_ref[...] = xs.astype(jnp.bfloat16)
        s_ref[...] = jnp.zeros((_B, 1), jnp.float32)

    u = lax.dot_general(
        xb_ref[...], feat_ref[...],
        (((1,), (1,)), ((), ())),
        preferred_element_type=jnp.float32,
    )

    @pl.when(i < _GRID - 1)
    def _accum():
        s_ref[...] += jnp.sum(jnp.exp2(u), axis=1, keepdims=True)

    @pl.when(i == _GRID - 1)
    def _finish():
        cols = i * _TILE + lax.broadcasted_iota(jnp.int32, (1, _TILE), 1)
        um = jnp.where(cols < _N, u, -jnp.inf)
        s = s_ref[...] + jnp.sum(jnp.exp2(um), axis=1, keepdims=True)
        u_sel = jnp.sum(xs_ref[...] * g_ref[...], axis=1, keepdims=True)
        loss_i = jnp.minimum(_LN2 * (jnp.log2(s) - u_sel), _CAP)
        out_ref[...] = jnp.sum(loss_i).reshape(1, 1) * (1.0 / _B)


def _tc_loss(inputs, g, features_bf16):
    return pl.pallas_call(
        _tc_body,
        grid=(_GRID,),
        in_specs=[
            pl.BlockSpec((_B, _D), lambda i: (0, 0)),
            pl.BlockSpec((_B, _D), lambda i: (0, 0)),
            pl.BlockSpec((_TILE, _D), lambda i: (i, 0)),
        ],
        out_specs=pl.BlockSpec((1, 1), lambda i: (0, 0)),
        out_shape=jax.ShapeDtypeStruct((1, 1), jnp.float32),
        scratch_shapes=[
            pltpu.VMEM((_B, _D), jnp.float32),
            pltpu.VMEM((_B, _D), jnp.bfloat16),
            pltpu.VMEM((_B, 1), jnp.float32),
        ],
    )(inputs, g, features_bf16)


def kernel(inputs, indexes, cameras, features):
    g = _sc_gather(features, indexes)
    out = _tc_loss(inputs, g, features.astype(jnp.bfloat16))
    return out[0, 0]


# speedup vs baseline: 6.6455x; 1.9712x over previous
import functools

import jax
import jax.numpy as jnp
from jax import lax
from jax.experimental import pallas as pl
from jax.experimental.pallas import tpu as pltpu, tpu_sc as plsc

_TEMP = 0.05
_B = 1024
_D = 128
_N = 100000
_TILE = 4096
_TFULL = _N // _TILE
_TAILBLK = 2048
_CAP = 18.420680743952367
_LOG2E = 1.4426950408889634
_LN2 = 0.6931471805599453


def _make_sc_gather():
    info = plsc.get_sparse_core_info()
    nc, ns = info.num_cores, info.num_subcores
    nw = nc * ns
    b_per_w = _B // nw
    mesh = plsc.VectorSubcoreMesh(core_axis_name="c", subcore_axis_name="s")

    @functools.partial(
        pl.kernel,
        mesh=mesh,
        out_type=jax.ShapeDtypeStruct((_B, _D), jnp.float32),
        scratch_types=[
            pltpu.VMEM((b_per_w,), jnp.int32),
            pltpu.VMEM((b_per_w, _D), jnp.float32),
            pltpu.SemaphoreType.DMA,
        ],
    )
    def sc_gather(table_hbm, idx_hbm, out_hbm, idx_v, rows_v, sem):
        wid = lax.axis_index("s") * nc + lax.axis_index("c")
        base = wid * b_per_w
        pltpu.sync_copy(idx_hbm.at[pl.ds(base, b_per_w)], idx_v)
        pltpu.async_copy(table_hbm.at[idx_v], rows_v, sem).wait()
        pltpu.sync_copy(rows_v, out_hbm.at[pl.ds(base, b_per_w)])

    return sc_gather


_sc_gather_cache = []


def _sc_gather(features, indexes):
    if not _sc_gather_cache:
        _sc_gather_cache.append(_make_sc_gather())
    return _sc_gather_cache[0](features, indexes)


_SUB = 2048
_NSUB = 4
_BLK = _SUB * _NSUB
_GSTEPS = _TFULL * _TILE // _BLK


def _lse_body(x_ref, feat_ref, out_ref, xb_ref, s_ref):
    i = pl.program_id(0)

    @pl.when(i == 0)
    def _init():
        x = x_ref[...]
        nrm = jnp.sqrt(jnp.sum(x * x, axis=1, keepdims=True))
        scale = _LOG2E / (jnp.maximum(nrm, 1e-12) * _TEMP)
        xb_ref[...] = (x * scale).astype(jnp.bfloat16)
        s_ref[...] = jnp.zeros((_B, 1), jnp.float32)

    xb = xb_ref[...]
    total = jnp.zeros((_B, 1), jnp.float32)
    for k in range(_NSUB):
        fk = feat_ref[pl.ds(k * _SUB, _SUB), :].astype(jnp.bfloat16)
        uk = lax.dot_general(
            xb, fk, (((1,), (1,)), ((), ())),
            preferred_element_type=jnp.float32,
        )
        total += jnp.sum(jnp.exp2(uk), axis=1, keepdims=True)
    s_ref[...] += total

    @pl.when(i == _GSTEPS - 1)
    def _emit():
        out_ref[...] = s_ref[...]


def _lse_sums(inputs, features):
    return pl.pallas_call(
        _lse_body,
        grid=(_GSTEPS,),
        in_specs=[
            pl.BlockSpec((_B, _D), lambda i: (0, 0)),
            pl.BlockSpec((_BLK, _D), lambda i: (i, 0)),
        ],
        out_specs=pl.BlockSpec((_B, 1), lambda i: (0, 0)),
        out_shape=jax.ShapeDtypeStruct((_B, 1), jnp.float32),
        scratch_shapes=[
            pltpu.VMEM((_B, _D), jnp.bfloat16),
            pltpu.VMEM((_B, 1), jnp.float32),
        ],
    )(inputs, features)


def _finish_body(x_ref, g_ref, tail_ref, s_in_ref, out_ref):
    x = x_ref[...]
    nrm = jnp.sqrt(jnp.sum(x * x, axis=1, keepdims=True))
    scale = _LOG2E / (jnp.maximum(nrm, 1e-12) * _TEMP)
    xs = x * scale
    ut = lax.dot_general(
        xs.astype(jnp.bfloat16), tail_ref[...].astype(jnp.bfloat16),
        (((1,), (1,)), ((), ())),
        preferred_element_type=jnp.float32,
    )
    cols = _TFULL * _TILE + lax.broadcasted_iota(jnp.int32, (1, _TAILBLK), 1)
    ut = jnp.where(cols < _N, ut, -jnp.inf)
    s_tot = s_in_ref[...] + jnp.sum(jnp.exp2(ut), axis=1, keepdims=True)
    u_sel = jnp.sum(xs * g_ref[...], axis=1, keepdims=True)
    loss_i = jnp.minimum(_LN2 * (jnp.log2(s_tot) - u_sel), _CAP)
    out_ref[...] = jnp.sum(loss_i).reshape(1, 1) * (1.0 / _B)


def _finish(inputs, g, features, s):
    tail_block = _TFULL * _TILE // _TAILBLK
    return pl.pallas_call(
        _finish_body,
        grid=(1,),
        in_specs=[
            pl.BlockSpec((_B, _D), lambda i: (0, 0)),
            pl.BlockSpec((_B, _D), lambda i: (0, 0)),
            pl.BlockSpec((_TAILBLK, _D), lambda i: (tail_block, 0)),
            pl.BlockSpec((_B, 1), lambda i: (0, 0)),
        ],
        out_specs=pl.BlockSpec((1, 1), lambda i: (0, 0)),
        out_shape=jax.ShapeDtypeStruct((1, 1), jnp.float32),
    )(inputs, g, features, s)


def kernel(inputs, indexes, cameras, features):
    g = _sc_gather(features, indexes)
    s = _lse_sums(inputs, features)
    out = _finish(inputs, g, features, s)
    return out[0, 0]


# speedup vs baseline: 6.7737x; 1.0193x over previous
import functools

import jax
import jax.numpy as jnp
from jax import lax
from jax.experimental import pallas as pl
from jax.experimental.pallas import tpu as pltpu, tpu_sc as plsc

_TEMP = 0.05
_B = 1024
_D = 128
_N = 100000
_TILE = 4096
_TFULL = _N // _TILE
_TAILBLK = 2048
_CAP = 18.420680743952367
_LOG2E = 1.4426950408889634
_LN2 = 0.6931471805599453


def _make_sc_gather():
    info = plsc.get_sparse_core_info()
    nc, ns = info.num_cores, info.num_subcores
    nw = nc * ns
    b_per_w = _B // nw
    mesh = plsc.VectorSubcoreMesh(core_axis_name="c", subcore_axis_name="s")

    @functools.partial(
        pl.kernel,
        mesh=mesh,
        out_type=jax.ShapeDtypeStruct((_B, _D), jnp.float32),
        scratch_types=[
            pltpu.VMEM((b_per_w,), jnp.int32),
            pltpu.VMEM((b_per_w, _D), jnp.float32),
            pltpu.SemaphoreType.DMA,
        ],
    )
    def sc_gather(table_hbm, idx_hbm, out_hbm, idx_v, rows_v, sem):
        wid = lax.axis_index("s") * nc + lax.axis_index("c")
        base = wid * b_per_w
        pltpu.sync_copy(idx_hbm.at[pl.ds(base, b_per_w)], idx_v)
        pltpu.async_copy(table_hbm.at[idx_v], rows_v, sem).wait()
        pltpu.sync_copy(rows_v, out_hbm.at[pl.ds(base, b_per_w)])

    return sc_gather


_sc_gather_cache = []


def _sc_gather(features, indexes):
    if not _sc_gather_cache:
        _sc_gather_cache.append(_make_sc_gather())
    return _sc_gather_cache[0](features, indexes)


_SUB = 4096
_NSUB = 4
_BLK = _SUB * _NSUB
_GSTEPS = _TFULL * _TILE // _BLK


def _lse_body(x_ref, feat_ref, out_ref, xb_ref, s_ref):
    i = pl.program_id(0)

    @pl.when(i == 0)
    def _init():
        x = x_ref[...]
        nrm = jnp.sqrt(jnp.sum(x * x, axis=1, keepdims=True))
        scale = _LOG2E / (jnp.maximum(nrm, 1e-12) * _TEMP)
        xb_ref[...] = (x * scale).astype(jnp.bfloat16)
        s_ref[...] = jnp.zeros((_B, 1), jnp.float32)

    xb = xb_ref[...]
    total = jnp.zeros((_B, 1), jnp.float32)
    for k in range(_NSUB):
        fk = feat_ref[pl.ds(k * _SUB, _SUB), :].astype(jnp.bfloat16)
        uk = lax.dot_general(
            xb, fk, (((1,), (1,)), ((), ())),
            preferred_element_type=jnp.float32,
        )
        total += jnp.sum(jnp.exp2(uk), axis=1, keepdims=True)
    s_ref[...] += total

    @pl.when(i == _GSTEPS - 1)
    def _emit():
        out_ref[...] = s_ref[...]


def _lse_sums(inputs, features):
    return pl.pallas_call(
        _lse_body,
        grid=(_GSTEPS,),
        in_specs=[
            pl.BlockSpec((_B, _D), lambda i: (0, 0)),
            pl.BlockSpec((_BLK, _D), lambda i: (i, 0)),
        ],
        out_specs=pl.BlockSpec((_B, 1), lambda i: (0, 0)),
        out_shape=jax.ShapeDtypeStruct((_B, 1), jnp.float32),
        scratch_shapes=[
            pltpu.VMEM((_B, _D), jnp.bfloat16),
            pltpu.VMEM((_B, 1), jnp.float32),
        ],
    )(inputs, features)


def _finish_body(x_ref, g_ref, tail_ref, s_in_ref, out_ref):
    x = x_ref[...]
    nrm = jnp.sqrt(jnp.sum(x * x, axis=1, keepdims=True))
    scale = _LOG2E / (jnp.maximum(nrm, 1e-12) * _TEMP)
    xs = x * scale
    ut = lax.dot_general(
        xs.astype(jnp.bfloat16), tail_ref[...].astype(jnp.bfloat16),
        (((1,), (1,)), ((), ())),
        preferred_element_type=jnp.float32,
    )
    cols = _TFULL * _TILE + lax.broadcasted_iota(jnp.int32, (1, _TAILBLK), 1)
    ut = jnp.where(cols < _N, ut, -jnp.inf)
    s_tot = s_in_ref[...] + jnp.sum(jnp.exp2(ut), axis=1, keepdims=True)
    u_sel = jnp.sum(xs * g_ref[...], axis=1, keepdims=True)
    loss_i = jnp.minimum(_LN2 * (jnp.log2(s_tot) - u_sel), _CAP)
    out_ref[...] = jnp.sum(loss_i).reshape(1, 1) * (1.0 / _B)


def _finish(inputs, g, features, s):
    tail_block = _TFULL * _TILE // _TAILBLK
    return pl.pallas_call(
        _finish_body,
        grid=(1,),
        in_specs=[
            pl.BlockSpec((_B, _D), lambda i: (0, 0)),
            pl.BlockSpec((_B, _D), lambda i: (0, 0)),
            pl.BlockSpec((_TAILBLK, _D), lambda i: (tail_block, 0)),
            pl.BlockSpec((_B, 1), lambda i: (0, 0)),
        ],
        out_specs=pl.BlockSpec((1, 1), lambda i: (0, 0)),
        out_shape=jax.ShapeDtypeStruct((1, 1), jnp.float32),
    )(inputs, g, features, s)


def kernel(inputs, indexes, cameras, features):
    g = _sc_gather(features, indexes)
    s = _lse_sums(inputs, features)
    out = _finish(inputs, g, features, s)
    return out[0, 0]
